# baseline (speedup 1.0000x reference)
import jax
import jax.numpy as jnp
from jax.experimental import pallas as pl
from jax.experimental.pallas import tpu as pltpu

_N_PAD = 1024
_NEG = -1e30
_PAIRS = 3


def _tc_kernel(feats_ref, sel0_ref, ch_ref, ids_ref,
               out_ref, v0t_ref, a2_ref, acc_ref):
    i = pl.program_id(0)
    jb = pl.program_id(1)
    n_blk = pl.num_programs(1)

    lane1000 = jax.lax.broadcasted_iota(jnp.int32, (1, 1000), 1)

    @pl.when(jb == 0)
    def _init():
        fi = feats_ref[pl.ds(i * 1000, 1000), :]
        sel0 = sel0_ref[0]
        onehot0 = (sel0 == lane1000).astype(jnp.float32)
        v0t = jax.lax.dot_general(
            fi, onehot0, (((0,), (1,)), ((), ())),
            precision=jax.lax.Precision.HIGHEST,
            preferred_element_type=jnp.float32)
        v0t_ref[...] = v0t
        a2_ref[...] = jnp.sum(v0t * v0t, axis=0, keepdims=True)
        acc_ref[...] = jnp.zeros_like(acc_ref)

    partials = []
    for k in range(_PAIRS):
        jj = jb * _PAIRS + k
        j = jj + (jj >= i).astype(jnp.int32)
        fj = feats_ref[pl.ds(j * 1000, 1000), :]
        ch = ch_ref[0, k]
        onehot1 = (ch == lane1000).astype(jnp.float32)
        v1 = jnp.dot(onehot1, fj, precision=jax.lax.Precision.HIGHEST,
                     preferred_element_type=jnp.float32)
        b2 = jnp.sum(v1 * v1, axis=1, keepdims=True)
        mt = jnp.dot(v1, v0t_ref[...],
                     preferred_element_type=jnp.float32)
        d2 = b2 + a2_ref[...] - 2.0 * mt
        dist = jnp.sqrt(jnp.maximum(d2, 0.0))
        partials.append(jnp.sum(dist, axis=0, keepdims=True))
    acc = acc_ref[...]
    for p in partials:
        acc = acc + p
    acc_ref[...] = acc

    @pl.when(jb == n_blk - 1)
    def _topk():
        lane = jax.lax.broadcasted_iota(jnp.int32, (1, _N_PAD), 1)
        mean = acc_ref[...] / jnp.float32(9000)
        mean = jnp.where(lane < 1000, mean, _NEG)
        ids_row = ids_ref[0]
        out_lane = jax.lax.broadcasted_iota(jnp.int32, (1, 128), 1)
        res = jnp.zeros((1, 128), jnp.int32)
        for k in range(100):
            m = jnp.max(mean)
            idx = jnp.min(jnp.where(mean == m, lane, jnp.int32(1 << 30)))
            idval = jnp.sum(jnp.where(lane == idx, ids_row, 0))
            res = res + jnp.where(out_lane == k, idval, 0)
            mean = jnp.where(lane == idx, _NEG, mean)
        out_ref[0] = res


def _build_indices(n_cls, per_cls, budget_dist_compute=1000):
    key = jax.random.key(42)
    sel0s, choices = [], []
    for i in range(n_cls):
        if per_cls < budget_dist_compute:
            sel0 = jnp.arange(per_cls, dtype=jnp.int32)
        else:
            key, ks = jax.random.split(key)
            sel0 = jax.random.randint(ks, (budget_dist_compute,), 0, per_cls)
        kk = min(budget_dist_compute, per_cls)
        chs = []
        for j in range(n_cls):
            if j == i:
                continue
            key, kc = jax.random.split(key)
            chs.append(jax.random.randint(kc, (kk,), 0, per_cls))
        sel0s.append(sel0)
        choices.append(jnp.stack(chs))
    return jnp.stack(sel0s), jnp.stack(choices)


def _run(ids_per_cls_train, budget, feats, interpret=False):
    n_cls, per_cls = ids_per_cls_train.shape
    sel0, choice = _build_indices(n_cls, per_cls)
    n_rows = sel0.shape[1]
    sel0_pad = jnp.pad(sel0, ((0, 0), (0, _N_PAD - n_rows)))
    sel0_pad = sel0_pad.astype(jnp.int32).reshape(n_cls, _N_PAD, 1)
    choice = choice.astype(jnp.int32).reshape(n_cls, n_cls - 1, per_cls, 1)
    ids_pad = jnp.pad(ids_per_cls_train.astype(jnp.int32),
                      ((0, 0), (0, _N_PAD - per_cls)))
    ids_pad = ids_pad.reshape(n_cls, 1, _N_PAD)

    grid = (n_cls, (n_cls - 1) // _PAIRS)
    out = pl.pallas_call(
        _tc_kernel,
        grid=grid,
        in_specs=[
            pl.BlockSpec((10000, 128), lambda i, jb: (0, 0)),
            pl.BlockSpec((1, _N_PAD, 1), lambda i, jb: (i, 0, 0)),
            pl.BlockSpec((1, _PAIRS, per_cls, 1), lambda i, jb: (i, jb, 0, 0)),
            pl.BlockSpec((1, 1, _N_PAD), lambda i, jb: (i, 0, 0)),
        ],
        out_specs=pl.BlockSpec((1, 1, 128), lambda i, jb: (i, 0, 0)),
        out_shape=jax.ShapeDtypeStruct((n_cls, 1, 128), jnp.int32),
        scratch_shapes=[
            pltpu.VMEM((128, _N_PAD), jnp.float32),
            pltpu.VMEM((1, _N_PAD), jnp.float32),
            pltpu.VMEM((1, _N_PAD), jnp.float32),
        ],
        interpret=interpret,
    )(feats, sel0_pad, choice, ids_pad)
    return out[:, 0, :100].reshape(-1)


def kernel(ids_per_cls_train, budget, feats):
    return _run(ids_per_cls_train, budget, feats)


# speedup vs baseline: 1.1583x; 1.1583x over previous
import functools

import jax
import jax.numpy as jnp
from jax import lax
from jax.experimental import pallas as pl
from jax.experimental.pallas import tpu as pltpu
from jax.experimental.pallas import tpu_sc as plsc

_N_PAD = 1024
_NEG = -1e30
_CH = 128


def _sc_gather_kernel(table_hbm, idx_hbm, out_hbm, idx_v, rows_v, sem):
    nc = plsc.get_sparse_core_info().num_cores
    wid = lax.axis_index("s") * nc + lax.axis_index("c")
    n_chunks = idx_hbm.shape[0] // _CH
    n_workers = nc * plsc.get_sparse_core_info().num_subcores
    per_w = n_chunks // n_workers

    def body(t, carry):
        base = (wid * per_w + t) * _CH
        pltpu.sync_copy(idx_hbm.at[pl.ds(base, _CH)], idx_v)
        pltpu.async_copy(table_hbm.at[idx_v], rows_v, sem).wait()
        pltpu.sync_copy(rows_v, out_hbm.at[pl.ds(base, _CH)])
        return carry

    lax.fori_loop(0, per_w, body, 0)


def _sc_gather(feats, idx_all):
    n_rows = idx_all.shape[0]
    mesh = plsc.VectorSubcoreMesh(core_axis_name="c", subcore_axis_name="s")
    k = functools.partial(
        pl.kernel,
        mesh=mesh,
        out_type=jax.ShapeDtypeStruct((n_rows, 128), jnp.float32),
        scratch_types=[
            pltpu.VMEM((_CH,), jnp.int32),
            pltpu.VMEM((_CH, 128), jnp.float32),
            pltpu.SemaphoreType.DMA,
        ],
    )(_sc_gather_kernel)
    return k(feats, idx_all)


def _tc_kernel(g0_ref, g1_ref, ids_ref, out_ref, v0t_ref, a2_ref, acc_ref):
    jj = pl.program_id(1)
    n_other = pl.num_programs(1)

    @pl.when(jj == 0)
    def _init():
        v0t = lax.transpose(g0_ref[...], (1, 0))
        v0t_ref[...] = v0t
        a2_ref[...] = jnp.sum(v0t * v0t, axis=0, keepdims=True)
        acc_ref[...] = jnp.zeros_like(acc_ref)

    v1 = g1_ref[0:1000, :]
    b2 = jnp.sum(v1 * v1, axis=1, keepdims=True)
    mt = jnp.dot(v1, v0t_ref[...],
                 preferred_element_type=jnp.float32)
    d2 = b2 + a2_ref[...] - 2.0 * mt
    dist = jnp.sqrt(jnp.maximum(d2, 0.0))
    acc_ref[...] += jnp.sum(dist, axis=0, keepdims=True)

    @pl.when(jj == n_other - 1)
    def _topk():
        lane = lax.broadcasted_iota(jnp.int32, (1, _N_PAD), 1)
        mean = acc_ref[...] / jnp.float32(n_other * 1000)
        mean = jnp.where(lane < 1000, mean, _NEG)
        ids_row = ids_ref[0]
        out_lane = lax.broadcasted_iota(jnp.int32, (1, 128), 1)
        res = jnp.zeros((1, 128), jnp.int32)
        for k in range(100):
            m = jnp.max(mean)
            idx = jnp.min(jnp.where(mean == m, lane, jnp.int32(1 << 30)))
            idval = jnp.sum(jnp.where(lane == idx, ids_row, 0))
            res = res + jnp.where(out_lane == k, idval, 0)
            mean = jnp.where(lane == idx, _NEG, mean)
        out_ref[0] = res


def _build_indices(n_cls, per_cls, budget_dist_compute=1000):
    key = jax.random.key(42)
    sel0s, choices = [], []
    for i in range(n_cls):
        if per_cls < budget_dist_compute:
            sel0 = jnp.arange(per_cls, dtype=jnp.int32)
        else:
            key, ks = jax.random.split(key)
            sel0 = jax.random.randint(ks, (budget_dist_compute,), 0, per_cls)
        kk = min(budget_dist_compute, per_cls)
        chs = []
        for j in range(n_cls):
            if j == i:
                continue
            key, kc = jax.random.split(key)
            chs.append(jax.random.randint(kc, (kk,), 0, per_cls))
        sel0s.append(sel0)
        choices.append(jnp.stack(chs))
    return jnp.stack(sel0s), jnp.stack(choices)


def kernel(ids_per_cls_train, budget, feats):
    n_cls, per_cls = ids_per_cls_train.shape
    ids32 = ids_per_cls_train.astype(jnp.int32)
    sel0, choice = _build_indices(n_cls, per_cls)

    pad = ((0, 0), (0, _N_PAD - per_cls))
    gid0 = jnp.take_along_axis(ids32, jnp.pad(sel0, pad), axis=1)
    jlists = jnp.asarray([[j for j in range(n_cls) if j != i]
                          for i in range(n_cls)], dtype=jnp.int32)
    ch_pad = jnp.pad(choice, ((0, 0), (0, 0), (0, _N_PAD - per_cls)))
    gid1 = jnp.take_along_axis(ids32[jlists.reshape(-1)],
                               ch_pad.reshape(-1, _N_PAD), axis=1)
    idx_all = jnp.concatenate([gid0.reshape(-1), gid1.reshape(-1)])

    gathered = _sc_gather(feats, idx_all)

    ids_pad = jnp.pad(ids32, pad).reshape(n_cls, 1, _N_PAD)

    grid = (n_cls, n_cls - 1)
    out = pl.pallas_call(
        _tc_kernel,
        grid=grid,
        in_specs=[
            pl.BlockSpec((_N_PAD, 128), lambda i, jj: (i, 0)),
            pl.BlockSpec((_N_PAD, 128),
                         lambda i, jj: (n_cls + i * (n_cls - 1) + jj, 0)),
            pl.BlockSpec((1, 1, _N_PAD), lambda i, jj: (i, 0, 0)),
        ],
        out_specs=pl.BlockSpec((1, 1, 128), lambda i, jj: (i, 0, 0)),
        out_shape=jax.ShapeDtypeStruct((n_cls, 1, 128), jnp.int32),
        scratch_shapes=[
            pltpu.VMEM((128, _N_PAD), jnp.float32),
            pltpu.VMEM((1, _N_PAD), jnp.float32),
            pltpu.VMEM((1, _N_PAD), jnp.float32),
        ],
    )(gathered, gathered, ids_pad)
    return out[:, 0, :100].reshape(-1)


# speedup vs baseline: 4.0689x; 3.5127x over previous
import functools

import jax
import jax.numpy as jnp
from jax import lax
from jax.experimental import pallas as pl
from jax.experimental.pallas import tpu as pltpu
from jax.experimental.pallas import tpu_sc as plsc

_N_PAD = 1024
_NEG = -1e30
_CH = 128


def _sc_gather_kernel(table_hbm, idx_hbm, out_hbm, idx_v, rows_v, sem):
    nc = plsc.get_sparse_core_info().num_cores
    wid = lax.axis_index("s") * nc + lax.axis_index("c")
    n_chunks = idx_hbm.shape[0] // _CH
    n_workers = nc * plsc.get_sparse_core_info().num_subcores
    per_w = n_chunks // n_workers

    def body(t, carry):
        base = (wid * per_w + t) * _CH
        pltpu.sync_copy(idx_hbm.at[pl.ds(base, _CH)], idx_v)
        pltpu.async_copy(table_hbm.at[idx_v], rows_v, sem).wait()
        pltpu.sync_copy(rows_v, out_hbm.at[pl.ds(base, _CH)])
        return carry

    lax.fori_loop(0, per_w, body, 0)


def _sc_gather(feats, idx_all):
    n_rows = idx_all.shape[0]
    mesh = plsc.VectorSubcoreMesh(core_axis_name="c", subcore_axis_name="s")
    k = functools.partial(
        pl.kernel,
        mesh=mesh,
        out_type=jax.ShapeDtypeStruct((n_rows, 128), jnp.float32),
        scratch_types=[
            pltpu.VMEM((_CH,), jnp.int32),
            pltpu.VMEM((_CH, 128), jnp.float32),
            pltpu.SemaphoreType.DMA,
        ],
    )(_sc_gather_kernel)
    return k(feats, idx_all)


def _tc_kernel(g0_ref, g1_ref, ids_ref, out_ref, v0t_ref, a2_ref, acc_ref):
    jj = pl.program_id(1)
    n_other = pl.num_programs(1)

    @pl.when(jj == 0)
    def _init():
        v0t = lax.transpose(g0_ref[...], (1, 0))
        v0t_ref[...] = v0t
        a2_ref[...] = jnp.sum(v0t * v0t, axis=0, keepdims=True)
        acc_ref[...] = jnp.zeros_like(acc_ref)

    v1 = g1_ref[0:1000, :]
    b2 = jnp.sum(v1 * v1, axis=1, keepdims=True)
    mt = jnp.dot(v1, v0t_ref[...],
                 preferred_element_type=jnp.float32)
    d2 = b2 + a2_ref[...] - 2.0 * mt
    dist = jnp.sqrt(jnp.maximum(d2, 0.0))
    acc_ref[...] += jnp.sum(dist, axis=0, keepdims=True)

    @pl.when(jj == n_other - 1)
    def _topk():
        lane = lax.broadcasted_iota(jnp.int32, (1, _N_PAD), 1)
        mean = acc_ref[...] / jnp.float32(n_other * 1000)
        mean = jnp.where(lane < 1000, mean, _NEG)
        ids_row = ids_ref[0]
        out_lane = lax.broadcasted_iota(jnp.int32, (1, 128), 1)
        res = jnp.zeros((1, 128), jnp.int32)
        for k in range(100):
            m = jnp.max(mean)
            idx = jnp.min(jnp.where(mean == m, lane, jnp.int32(1 << 30)))
            idval = jnp.sum(jnp.where(lane == idx, ids_row, 0))
            res = res + jnp.where(out_lane == k, idval, 0)
            mean = jnp.where(lane == idx, _NEG, mean)
        out_ref[0] = res


@functools.lru_cache(maxsize=None)
def _build_indices_host(n_cls, per_cls):
    import numpy as np
    with jax.ensure_compile_time_eval():
        with jax.default_device(jax.devices("cpu")[0]):
            sel0, choice = _build_indices(n_cls, per_cls)
            return np.asarray(sel0), np.asarray(choice)


def _build_indices(n_cls, per_cls, budget_dist_compute=1000):
    key = jax.random.key(42)
    sel0s, choices = [], []
    for i in range(n_cls):
        if per_cls < budget_dist_compute:
            sel0 = jnp.arange(per_cls, dtype=jnp.int32)
        else:
            key, ks = jax.random.split(key)
            sel0 = jax.random.randint(ks, (budget_dist_compute,), 0, per_cls)
        kk = min(budget_dist_compute, per_cls)
        chs = []
        for j in range(n_cls):
            if j == i:
                continue
            key, kc = jax.random.split(key)
            chs.append(jax.random.randint(kc, (kk,), 0, per_cls))
        sel0s.append(sel0)
        choices.append(jnp.stack(chs))
    return jnp.stack(sel0s), jnp.stack(choices)


def kernel(ids_per_cls_train, budget, feats):
    n_cls, per_cls = ids_per_cls_train.shape
    ids32 = ids_per_cls_train.astype(jnp.int32)
    sel0_np, choice_np = _build_indices_host(n_cls, per_cls)
    sel0 = jnp.asarray(sel0_np)
    choice = jnp.asarray(choice_np)

    pad = ((0, 0), (0, _N_PAD - per_cls))
    gid0 = jnp.take_along_axis(ids32, jnp.pad(sel0, pad), axis=1)
    jlists = jnp.asarray([[j for j in range(n_cls) if j != i]
                          for i in range(n_cls)], dtype=jnp.int32)
    ch_pad = jnp.pad(choice, ((0, 0), (0, 0), (0, _N_PAD - per_cls)))
    gid1 = jnp.take_along_axis(ids32[jlists.reshape(-1)],
                               ch_pad.reshape(-1, _N_PAD), axis=1)
    idx_all = jnp.concatenate([gid0.reshape(-1), gid1.reshape(-1)])

    gathered = _sc_gather(feats, idx_all)

    ids_pad = jnp.pad(ids32, pad).reshape(n_cls, 1, _N_PAD)

    grid = (n_cls, n_cls - 1)
    out = pl.pallas_call(
        _tc_kernel,
        grid=grid,
        in_specs=[
            pl.BlockSpec((_N_PAD, 128), lambda i, jj: (i, 0)),
            pl.BlockSpec((_N_PAD, 128),
                         lambda i, jj: (n_cls + i * (n_cls - 1) + jj, 0)),
            pl.BlockSpec((1, 1, _N_PAD), lambda i, jj: (i, 0, 0)),
        ],
        out_specs=pl.BlockSpec((1, 1, 128), lambda i, jj: (i, 0, 0)),
        out_shape=jax.ShapeDtypeStruct((n_cls, 1, 128), jnp.int32),
        scratch_shapes=[
            pltpu.VMEM((128, _N_PAD), jnp.float32),
            pltpu.VMEM((1, _N_PAD), jnp.float32),
            pltpu.VMEM((1, _N_PAD), jnp.float32),
        ],
    )(gathered, gathered, ids_pad)
    return out[:, 0, :100].reshape(-1)


# speedup vs baseline: 7.7355x; 1.9011x over previous
import functools

import jax
import jax.numpy as jnp
from jax import lax
from jax.experimental import pallas as pl
from jax.experimental.pallas import tpu as pltpu
from jax.experimental.pallas import tpu_sc as plsc

_N_PAD = 1024
_NEG = -1e30
_CH = 128


def _sc_gather_kernel(table_hbm, idx_hbm, out_hbm, idx_v, rows_v, sem):
    nc = plsc.get_sparse_core_info().num_cores
    wid = lax.axis_index("s") * nc + lax.axis_index("c")
    n_chunks = idx_hbm.shape[0] // _CH
    n_workers = nc * plsc.get_sparse_core_info().num_subcores
    per_w = n_chunks // n_workers

    def body(t, carry):
        base = (wid * per_w + t) * _CH
        pltpu.sync_copy(idx_hbm.at[pl.ds(base, _CH)], idx_v)
        pltpu.async_copy(table_hbm.at[idx_v], rows_v, sem).wait()
        pltpu.sync_copy(rows_v, out_hbm.at[pl.ds(base, _CH)])
        return carry

    lax.fori_loop(0, per_w, body, 0)


def _sc_gather(feats, idx_all):
    n_rows = idx_all.shape[0]
    mesh = plsc.VectorSubcoreMesh(core_axis_name="c", subcore_axis_name="s")
    k = functools.partial(
        pl.kernel,
        mesh=mesh,
        out_type=jax.ShapeDtypeStruct((n_rows, 128), jnp.float32),
        scratch_types=[
            pltpu.VMEM((_CH,), jnp.int32),
            pltpu.VMEM((_CH, 128), jnp.float32),
            pltpu.SemaphoreType.DMA,
        ],
    )(_sc_gather_kernel)
    return k(feats, idx_all)


def _tc_kernel(g0_ref, g1_ref, out_ref, v0t_ref, a2_ref, acc_ref):
    jj = pl.program_id(1)
    n_other = pl.num_programs(1)

    @pl.when(jj == 0)
    def _init():
        v0t = lax.transpose(g0_ref[...], (1, 0))
        v0t_ref[...] = v0t
        a2_ref[...] = jnp.sum(v0t * v0t, axis=0, keepdims=True)
        acc_ref[...] = jnp.zeros_like(acc_ref)

    v1 = g1_ref[0:1000, :]
    b2 = jnp.sum(v1 * v1, axis=1, keepdims=True)
    mt = jnp.dot(v1, v0t_ref[...],
                 preferred_element_type=jnp.float32)
    d2 = b2 + a2_ref[...] - 2.0 * mt
    dist = jnp.sqrt(jnp.maximum(d2, 0.0))
    acc_ref[...] += jnp.sum(dist, axis=0, keepdims=True)

    @pl.when(jj == n_other - 1)
    def _emit_mean():
        lane = lax.broadcasted_iota(jnp.int32, (1, _N_PAD), 1)
        mean = acc_ref[...] / jnp.float32(n_other * 1000)
        out_ref[0] = jnp.where(lane < 1000, mean, _NEG)


def _topk_kernel(mean_ref, ids_ref, out_ref):
    n_cls = mean_ref.shape[0]
    mean = mean_ref[...]
    ids_m = ids_ref[...]
    lane = lax.broadcasted_iota(jnp.int32, (n_cls, _N_PAD), 1)
    out_lane = lax.broadcasted_iota(jnp.int32, (n_cls, 128), 1)
    res = jnp.zeros((n_cls, 128), jnp.int32)
    for k in range(100):
        m = jnp.max(mean, axis=1, keepdims=True)
        idx = jnp.min(jnp.where(mean == m, lane, jnp.int32(1 << 30)),
                      axis=1, keepdims=True)
        hit = lane == idx
        idval = jnp.sum(jnp.where(hit, ids_m, 0), axis=1, keepdims=True)
        res = res + jnp.where(out_lane == k, idval, 0)
        mean = jnp.where(hit, _NEG, mean)
    out_ref[...] = res


@functools.lru_cache(maxsize=None)
def _build_indices_host(n_cls, per_cls):
    import numpy as np
    with jax.ensure_compile_time_eval():
        with jax.default_device(jax.devices("cpu")[0]):
            sel0, choice = _build_indices(n_cls, per_cls)
            return np.asarray(sel0), np.asarray(choice)


def _build_indices(n_cls, per_cls, budget_dist_compute=1000):
    key = jax.random.key(42)
    sel0s, choices = [], []
    for i in range(n_cls):
        if per_cls < budget_dist_compute:
            sel0 = jnp.arange(per_cls, dtype=jnp.int32)
        else:
            key, ks = jax.random.split(key)
            sel0 = jax.random.randint(ks, (budget_dist_compute,), 0, per_cls)
        kk = min(budget_dist_compute, per_cls)
        chs = []
        for j in range(n_cls):
            if j == i:
                continue
            key, kc = jax.random.split(key)
            chs.append(jax.random.randint(kc, (kk,), 0, per_cls))
        sel0s.append(sel0)
        choices.append(jnp.stack(chs))
    return jnp.stack(sel0s), jnp.stack(choices)


def kernel(ids_per_cls_train, budget, feats):
    n_cls, per_cls = ids_per_cls_train.shape
    ids32 = ids_per_cls_train.astype(jnp.int32)
    sel0_np, choice_np = _build_indices_host(n_cls, per_cls)
    sel0 = jnp.asarray(sel0_np)
    choice = jnp.asarray(choice_np)

    pad = ((0, 0), (0, _N_PAD - per_cls))
    gid0 = jnp.take_along_axis(ids32, jnp.pad(sel0, pad), axis=1)
    jlists = jnp.asarray([[j for j in range(n_cls) if j != i]
                          for i in range(n_cls)], dtype=jnp.int32)
    ch_pad = jnp.pad(choice, ((0, 0), (0, 0), (0, _N_PAD - per_cls)))
    gid1 = jnp.take_along_axis(ids32[jlists.reshape(-1)],
                               ch_pad.reshape(-1, _N_PAD), axis=1)
    idx_all = jnp.concatenate([gid0.reshape(-1), gid1.reshape(-1)])

    gathered = _sc_gather(feats, idx_all)

    grid = (n_cls, n_cls - 1)
    means = pl.pallas_call(
        _tc_kernel,
        grid=grid,
        in_specs=[
            pl.BlockSpec((_N_PAD, 128), lambda i, jj: (i, 0)),
            pl.BlockSpec((_N_PAD, 128),
                         lambda i, jj: (n_cls + i * (n_cls - 1) + jj, 0)),
        ],
        out_specs=pl.BlockSpec((1, 1, _N_PAD), lambda i, jj: (i, 0, 0)),
        out_shape=jax.ShapeDtypeStruct((n_cls, 1, _N_PAD), jnp.float32),
        scratch_shapes=[
            pltpu.VMEM((128, _N_PAD), jnp.float32),
            pltpu.VMEM((1, _N_PAD), jnp.float32),
            pltpu.VMEM((1, _N_PAD), jnp.float32),
        ],
    )(gathered, gathered)

    ids_pad = jnp.pad(ids32, pad)
    out = pl.pallas_call(
        _topk_kernel,
        out_shape=jax.ShapeDtypeStruct((n_cls, 128), jnp.int32),
    )(means.reshape(n_cls, _N_PAD), ids_pad)
    return out[:, :100].reshape(-1)
